# baseline (speedup 1.0000x reference)
import functools

import jax
import jax.numpy as jnp
from jax import lax
from jax.experimental import pallas as pl
from jax.experimental.pallas import tpu as pltpu
from jax.experimental.pallas import tpu_sc as plsc

B = 4096
L = 200
D = 64
NUM_WORKERS = 32
RPW = B // NUM_WORKERS
HALF = L // 2
LANES = 16
NCHUNK = D // LANES


def _worker_id():
    return lax.axis_index("s") * 2 + lax.axis_index("c")


def _body(tokens_hbm, len_hbm, table_hbm, out_hbm,
          idx_v, len_v, inv_v, buf, outw, sem):
    wid = _worker_id()

    pltpu.sync_copy(tokens_hbm.at[wid], idx_v)
    pltpu.sync_copy(len_hbm.at[wid], len_v)

    for g in range(RPW // LANES):
        lens16 = len_v[pl.ds(g * LANES, LANES)]
        inv_v[pl.ds(g * LANES, LANES)] = (
            1.0 / jnp.maximum(lens16, 1).astype(jnp.float32))

    lane = lax.broadcasted_iota(jnp.int32, (LANES,), 0)

    def row_body(r, carry):
        cp0 = pltpu.async_copy(table_hbm.at[idx_v.at[r, 0]],
                               buf.at[pl.ds(0, HALF)], sem)
        cp1 = pltpu.async_copy(table_hbm.at[idx_v.at[r, 1]],
                               buf.at[pl.ds(HALF, HALF)], sem)
        cp0.wait()
        cp1.wait()

        def acc_body(t, accs):
            return tuple(a + buf[t, pl.ds(c * LANES, LANES)]
                         for c, a in enumerate(accs))

        accs = lax.fori_loop(
            0, L, acc_body,
            tuple(jnp.zeros((LANES,), jnp.float32) for _ in range(NCHUNK)))

        sinv = plsc.load_gather(inv_v, [lane * 0 + r])
        for c in range(NCHUNK):
            outw[r, pl.ds(c * LANES, LANES)] = accs[c] * sinv
        return carry

    lax.fori_loop(0, RPW, row_body, 0)
    pltpu.sync_copy(outw, out_hbm.at[pl.ds(wid * RPW, RPW)])


@functools.partial(jax.jit, static_argnames=("interpret",))
def _run(tokens, token_lengths, table, interpret=False):
    mesh = plsc.VectorSubcoreMesh(core_axis_name="c", subcore_axis_name="s",
                                  num_cores=2, num_subcores=16)
    tok = tokens.reshape(NUM_WORKERS, RPW, 2, HALF)
    lens = token_lengths.reshape(NUM_WORKERS, RPW)
    f = pl.kernel(
        _body,
        out_type=jax.ShapeDtypeStruct((B, D), jnp.float32),
        mesh=mesh,
        compiler_params=pltpu.CompilerParams(needs_layout_passes=False,
                                             use_tc_tiling_on_sc=False),
        scratch_types=[
            pltpu.VMEM((RPW, 2, HALF), jnp.int32),
            pltpu.VMEM((RPW,), jnp.int32),
            pltpu.VMEM((RPW,), jnp.float32),
            pltpu.VMEM((L, D), jnp.float32),
            pltpu.VMEM((RPW, D), jnp.float32),
            pltpu.SemaphoreType.DMA,
        ],
        interpret=interpret,
    )
    return f(tok, lens, table)


def kernel(tokens, token_lengths, table):
    return _run(tokens, token_lengths, table)


# speedup vs baseline: 1.1668x; 1.1668x over previous
import functools

import jax
import jax.numpy as jnp
from jax import lax
from jax.experimental import pallas as pl
from jax.experimental.pallas import tpu as pltpu
from jax.experimental.pallas import tpu_sc as plsc

B = 4096
L = 200
D = 64
NUM_WORKERS = 32
RPW = B // NUM_WORKERS
HALF = L // 2
LANES = 16
NCHUNK = D // LANES


def _worker_id():
    return lax.axis_index("s") * 2 + lax.axis_index("c")


def _body(tokens_hbm, len_hbm, table_hbm, out_hbm,
          idx_v, len_v, inv_v, buf, outw, sems):
    wid = _worker_id()

    pltpu.sync_copy(tokens_hbm.at[wid], idx_v)
    pltpu.sync_copy(len_hbm.at[wid], len_v)

    for g in range(RPW // LANES):
        lens16 = len_v[pl.ds(g * LANES, LANES)]
        inv_v[pl.ds(g * LANES, LANES)] = (
            1.0 / jnp.maximum(lens16, 1).astype(jnp.float32))

    lane = lax.broadcasted_iota(jnp.int32, (LANES,), 0)
    sem0, sem1 = sems

    def issue(r, slot, sem):
        pltpu.async_copy(table_hbm.at[idx_v.at[r, 0]],
                         buf.at[slot, pl.ds(0, HALF)], sem)
        pltpu.async_copy(table_hbm.at[idx_v.at[r, 1]],
                         buf.at[slot, pl.ds(HALF, HALF)], sem)

    def drain(slot, sem):
        pltpu.make_async_copy(table_hbm.at[pl.ds(0, L)],
                              buf.at[slot], sem).wait()

    def accumulate(r, slot):
        def acc_body(i, accs):
            t = i * 2
            new = []
            for c in range(NCHUNK):
                new.append(accs[c] + buf[slot, t, pl.ds(c * LANES, LANES)])
            for c in range(NCHUNK):
                new.append(accs[NCHUNK + c]
                           + buf[slot, t + 1, pl.ds(c * LANES, LANES)])
            return tuple(new)

        accs = lax.fori_loop(
            0, L // 2, acc_body,
            tuple(jnp.zeros((LANES,), jnp.float32)
                  for _ in range(2 * NCHUNK)),
            unroll=4)

        sinv = plsc.load_gather(inv_v, [lane * 0 + r])
        for c in range(NCHUNK):
            outw[r, pl.ds(c * LANES, LANES)] = (
                (accs[c] + accs[NCHUNK + c]) * sinv)

    issue(0, 0, sem0)

    def pair_body(p, carry):
        r0 = 2 * p
        r1 = r0 + 1
        issue(r1, 1, sem1)
        drain(0, sem0)
        accumulate(r0, 0)
        issue(jnp.minimum(r1 + 1, RPW - 1), 0, sem0)
        drain(1, sem1)
        accumulate(r1, 1)
        return carry

    lax.fori_loop(0, RPW // 2, pair_body, 0)
    drain(0, sem0)
    pltpu.sync_copy(outw, out_hbm.at[pl.ds(wid * RPW, RPW)])


@functools.partial(jax.jit, static_argnames=("interpret",))
def _run(tokens, token_lengths, table, interpret=False):
    mesh = plsc.VectorSubcoreMesh(core_axis_name="c", subcore_axis_name="s",
                                  num_cores=2, num_subcores=16)
    tok = tokens.reshape(NUM_WORKERS, RPW, 2, HALF)
    lens = token_lengths.reshape(NUM_WORKERS, RPW)
    f = pl.kernel(
        _body,
        out_type=jax.ShapeDtypeStruct((B, D), jnp.float32),
        mesh=mesh,
        compiler_params=pltpu.CompilerParams(needs_layout_passes=False,
                                             use_tc_tiling_on_sc=False),
        scratch_types=[
            pltpu.VMEM((RPW, 2, HALF), jnp.int32),
            pltpu.VMEM((RPW,), jnp.int32),
            pltpu.VMEM((RPW,), jnp.float32),
            pltpu.VMEM((2, L, D), jnp.float32),
            pltpu.VMEM((RPW, D), jnp.float32),
            (pltpu.SemaphoreType.DMA, pltpu.SemaphoreType.DMA),
        ],
        interpret=interpret,
    )
    return f(tok, lens, table)


def kernel(tokens, token_lengths, table):
    return _run(tokens, token_lengths, table)
